# baseline (speedup 1.0000x reference)
import jax
import jax.numpy as jnp
from jax.experimental import pallas as pl
from jax.experimental.pallas import tpu as pltpu

_B, _N, _C_IN, _C_OUT = 4, 1024, 768, 384
_G = _N // 8


def _stats(xn):
    g = xn.reshape(_G, 8, _C_OUT)
    rsum = jnp.sum(g, axis=2)
    p2a = jnp.sum(g[:, 2, 0:256], axis=1, keepdims=True)
    p5a = jnp.sum(g[:, 5, 0:128], axis=1, keepdims=True)
    s0 = rsum[:, 0:1] + rsum[:, 1:2] + p2a
    s1 = (rsum[:, 2:3] - p2a) + rsum[:, 3:4] + rsum[:, 4:5] + p5a
    s2 = (rsum[:, 5:6] - p5a) + rsum[:, 6:7] + rsum[:, 7:8]
    return s0, s1, s2


def _body(x_ref, theta_ref, bias_ref, w_ref, b_ref, out_ref, acc_ref):
    b = pl.program_id(0)

    @pl.when(b == 0)
    def _zero():
        acc_ref[...] = jnp.zeros_like(acc_ref)

    xb = x_ref[0]
    theta = theta_ref[...]
    bias = bias_ref[...]

    inner = -2.0 * jnp.dot(xb, xb.T)
    sq = jnp.sum(xb * xb, axis=1, keepdims=True)
    dis = sq + inner + sq.T

    col = jax.lax.broadcasted_iota(jnp.int32, (_N, _N), 1)
    row = jax.lax.broadcasted_iota(jnp.int32, (_N, _N), 0)

    m1 = jnp.min(dis, axis=1, keepdims=True)
    i1 = jnp.min(jnp.where(dis == m1, col, _N), axis=1, keepdims=True)
    dis2 = jnp.where(col == i1, jnp.inf, dis)
    m2 = jnp.min(dis2, axis=1, keepdims=True)
    i2 = jnp.min(jnp.where(dis2 == m2, col, _N), axis=1, keepdims=True)

    h = ((row == i1.T) | (row == i2.T) | (row == col)).astype(jnp.float32)
    ht = ((col == i1) | (col == i2) | (col == row)).astype(jnp.float32)

    rowvec = jax.lax.broadcasted_iota(jnp.int32, (_N, 1), 0)
    de = (3.0
          - (i1 == rowvec).astype(jnp.float32)
          - (i2 == rowvec).astype(jnp.float32))

    xt = jnp.dot(xb, theta)
    xe = jnp.dot(ht, xt) / de
    dn = jnp.sum(h, axis=1, keepdims=True)
    xn = jnp.dot(h, xe) / dn + bias

    out_ref[b] = xn

    s0, s1, s2 = _stats(xn)
    q0, q1, q2 = _stats(xn * xn)
    acc_ref[:, 0:1] += s0
    acc_ref[:, 1:2] += s1
    acc_ref[:, 2:3] += s2
    acc_ref[:, 4:5] += q0
    acc_ref[:, 5:6] += q1
    acc_ref[:, 6:7] += q2

    @pl.when(b == _B - 1)
    def _normalize():
        cnt = jnp.float32(_B * _N)
        w2 = w_ref[...]
        b2 = b_ref[...]
        r3 = jax.lax.broadcasted_iota(jnp.int32, (_G, 8, _C_OUT), 1)
        c3 = jax.lax.broadcasted_iota(jnp.int32, (_G, 8, _C_OUT), 2)
        ch = (r3 * _C_OUT + c3) // _N

        def full(t):
            mean = acc_ref[:, t:t + 1] / cnt
            var = acc_ref[:, t + 4:t + 5] / cnt - mean * mean
            scale = w2[:, t:t + 1] / jnp.sqrt(var + 1e-5)
            shift = b2[:, t:t + 1] - mean * scale
            return scale[:, :, None], shift[:, :, None]

        sc0, sh0 = full(0)
        sc1, sh1 = full(1)
        sc2, sh2 = full(2)
        scalef = jnp.where(ch == 0, sc0, jnp.where(ch == 1, sc1, sc2))
        shiftf = jnp.where(ch == 0, sh0, jnp.where(ch == 1, sh1, sh2))
        for bb in range(_B):
            v = out_ref[bb].reshape(_G, 8, _C_OUT)
            y = jnp.maximum(v * scalef + shiftf, 0.0)
            out_ref[bb] = y.reshape(_N, _C_OUT)


def kernel(x, theta, bias, bn_weight, bn_bias):
    return pl.pallas_call(
        _body,
        grid=(_B,),
        in_specs=[
            pl.BlockSpec((1, _N, _C_IN), lambda b: (b, 0, 0)),
            pl.BlockSpec((_C_IN, _C_OUT), lambda b: (0, 0)),
            pl.BlockSpec((1, _C_OUT), lambda b: (0, 0)),
            pl.BlockSpec((_G, 3), lambda b: (0, 0)),
            pl.BlockSpec((_G, 3), lambda b: (0, 0)),
        ],
        out_specs=pl.BlockSpec((_B, _N, _C_OUT), lambda b: (0, 0, 0)),
        out_shape=jax.ShapeDtypeStruct((_B, _N, _C_OUT), jnp.float32),
        scratch_shapes=[pltpu.VMEM((_G, 8), jnp.float32)],
    )(x, theta, bias.reshape(1, _C_OUT),
      bn_weight.reshape(_G, 3), bn_bias.reshape(_G, 3))
